# Initial kernel scaffold; baseline (speedup 1.0000x reference)
import jax
import jax.numpy as jnp
from jax.experimental import pallas as pl


def kernel(x, ema_vals):
    raise NotImplementedError("write your pallas kernel here")


# speedup vs baseline: 27.1383x; 27.1383x over previous
import functools

import jax
import jax.numpy as jnp
from jax import lax
from jax.experimental import pallas as pl
from jax.experimental.pallas import tpu as pltpu

_INT_MIN = -2147483648
_INT_MAX = 2147483647


def _skey(xbits):
    return jnp.where(xbits >= 0, xbits, jnp.int32(_INT_MIN) - xbits)


def _skey_to_float(k):
    return lax.bitcast_convert_type(
        jnp.where(k >= 0, k, jnp.int32(_INT_MIN) - k), jnp.float32)


def _select_kernel(x_ref, ema_ref, out0_ref, out1_ref, key_ref, *, n, r1, f1,
                   r2, f2, alpha):
    xbits = lax.bitcast_convert_type(x_ref[...], jnp.int32)
    key_ref[...] = _skey(xbits)

    def bit_step(t, carry):
        p1, p2 = carry
        b = 31 - t
        bit = jnp.left_shift(jnp.int32(1), b)
        k = key_ref[...]
        c1 = jnp.sum((k < (p1 + bit)).astype(jnp.int32))
        c2 = jnp.sum((k < (p2 + bit)).astype(jnp.int32))
        p1 = jnp.where(c1 <= r1, p1 + bit, p1)
        p2 = jnp.where(c2 <= r2, p2 + bit, p2)
        return p1, p2

    p1, p2 = lax.fori_loop(
        0, 32, bit_step, (jnp.int32(_INT_MIN), jnp.int32(_INT_MIN)))

    k = key_ref[...]
    cle1 = jnp.sum((k <= p1).astype(jnp.int32))
    cle2 = jnp.sum((k <= p2).astype(jnp.int32))
    ma1 = jnp.min(jnp.where(k > p1, k, jnp.int32(_INT_MAX)))
    ma2 = jnp.min(jnp.where(k > p2, k, jnp.int32(_INT_MAX)))
    p1b = jnp.where(cle1 >= r1 + 2, p1, ma1)
    p2b = jnp.where(cle2 >= r2 + 2, p2, ma2)

    v1 = _skey_to_float(p1)
    v1b = _skey_to_float(p1b)
    v2 = _skey_to_float(p2)
    v2b = _skey_to_float(p2b)
    q1 = v1 + jnp.float32(f1) * (v1b - v1)
    q2 = v2 + jnp.float32(f2) * (v2b - v2)

    new0 = jnp.float32(alpha) * q1 + jnp.float32(1.0 - alpha) * ema_ref[0, 0]
    new1 = jnp.float32(alpha) * q2 + jnp.float32(1.0 - alpha) * ema_ref[1, 0]
    offset = new0
    scale = jnp.maximum(new1 - new0, jnp.float32(1.0))
    out0_ref[...] = jnp.full((1, 128), offset, jnp.float32)
    out1_ref[...] = jnp.full((1, 128), scale, jnp.float32)


def kernel(x, ema_vals):
    alpha = 0.01
    n = x.size
    pos1 = 0.05 * (n - 1)
    pos2 = 0.95 * (n - 1)
    r1, f1 = int(pos1), pos1 - int(pos1)
    r2, f2 = int(pos2), pos2 - int(pos2)

    ema2d = jnp.broadcast_to(ema_vals[:, None], (2, 128))
    body = functools.partial(_select_kernel, n=n, r1=r1, f1=f1, r2=r2, f2=f2,
                             alpha=alpha)
    out0, out1 = pl.pallas_call(
        body,
        out_shape=(jax.ShapeDtypeStruct((1, 128), jnp.float32),
                   jax.ShapeDtypeStruct((1, 128), jnp.float32)),
        scratch_shapes=[pltpu.VMEM(x.shape, jnp.int32)],
    )(x, ema2d)
    return out0[0, 0], out1[0, 0]
